# baseline (speedup 1.0000x reference)
import functools

import jax
import jax.numpy as jnp
from jax.experimental import pallas as pl
from jax.experimental.pallas import tpu as pltpu

H = 768
I = 4096
E = 16
V = 32000
EIS = I // E
T = 512
CH = 256
RP = 1024
NBUF = 8


def _moe_kernel(tid_ref, xh_ref, muh_ref, w_ref, gup_ref, dnp_ref, out_ref,
                x_ref, mu_ref, r_ref, xg_ref, og_ref, gu_buf, dn_buf,
                plan_ref, psmem_ref, gu_sem, dn_sem, x_sem, mu_sem, p_sem):
    def start(e):
        slot = jax.lax.rem(e, NBUF)
        pltpu.make_async_copy(
            gup_ref.at[e], gu_buf.at[slot], gu_sem.at[slot]).start()
        pltpu.make_async_copy(
            dnp_ref.at[e], dn_buf.at[slot], dn_sem.at[slot]).start()

    x_cp = pltpu.make_async_copy(xh_ref, x_ref, x_sem)
    mu_cp = pltpu.make_async_copy(muh_ref, mu_ref, mu_sem)
    x_cp.start()
    mu_cp.start()
    for e in range(NBUF):
        start(e)

    mu_cp.wait()
    logits = jax.lax.dot_general(
        mu_ref[...], w_ref[...],
        dimension_numbers=(((1,), (1,)), ((), ())),
        preferred_element_type=jnp.float32,
    )
    tid = tid_ref[...]
    base_e = jnp.bitwise_and(jnp.clip(tid, 0, V - 1), E - 1)
    iota_e = jax.lax.broadcasted_iota(jnp.int32, (T, E), 1)
    onehot_f = (base_e == iota_e).astype(jnp.float32)
    combined = onehot_f * 10.0 + logits
    m = jnp.max(combined, axis=-1, keepdims=True)
    eid = jnp.min(jnp.where(combined == m, iota_e, E), axis=-1, keepdims=True)

    onehot = (iota_e == eid).astype(jnp.int32)
    iota_t = jax.lax.broadcasted_iota(jnp.int32, (T, E), 0)
    cum = onehot
    d = 1
    while d < T:
        rolled = pltpu.roll(cum, d, 0)
        cum = cum + jnp.where(iota_t >= d, rolled, 0)
        d *= 2
    rank = jnp.sum(cum * onehot, axis=-1, keepdims=True) - 1
    counts = cum[T - 1:T, :]
    ctp = jnp.bitwise_and(counts + 15, ~15)
    iota_l = jax.lax.broadcasted_iota(jnp.int32, (1, E), 1)
    coff = ctp
    d = 1
    while d < E:
        rolled = pltpu.roll(coff, d, 1)
        coff = coff + jnp.where(iota_l >= d, rolled, 0)
        d *= 2
    off = coff - ctp
    off_tok = jnp.sum(off * onehot, axis=-1, keepdims=True)
    row = off_tok + rank

    plan_ref[0:1, :] = jnp.right_shift(off, 4)
    plan_ref[1:2, :] = ctp
    p_cp = pltpu.make_async_copy(plan_ref, psmem_ref, p_sem)
    p_cp.start()

    iota_r = jax.lax.broadcasted_iota(jnp.int32, (T, RP), 1)
    r_ref[...] = (iota_r == row).astype(jnp.bfloat16)
    x_cp.wait()
    xg = jax.lax.dot_general(
        r_ref[...], x_ref[...].astype(jnp.bfloat16),
        dimension_numbers=(((0,), (0,)), ((), ())),
        preferred_element_type=jnp.float32,
    )
    xg_ref[...] = xg.astype(jnp.bfloat16)
    og_ref[...] = jnp.zeros((RP, H), jnp.bfloat16)
    p_cp.wait()

    def expert_body(e, carry):
        slot = jax.lax.rem(e, NBUF)
        pltpu.make_async_copy(
            gup_ref.at[e], gu_buf.at[slot], gu_sem.at[slot]).wait()
        pltpu.make_async_copy(
            dnp_ref.at[e], dn_buf.at[slot], dn_sem.at[slot]).wait()
        gu_bf = gu_buf[slot].astype(jnp.bfloat16)
        dn_bf = dn_buf[slot].astype(jnp.bfloat16)

        def chunk(s):
            rbase = (psmem_ref[0, e] + s * (CH // 16)) * 16
            xchunk = xg_ref[pl.ds(rbase, CH), :]
            h = jax.lax.dot_general(
                xchunk, gu_bf,
                dimension_numbers=(((1,), (0,)), ((), ())),
                preferred_element_type=jnp.float32,
            )
            gate = h[:, :EIS]
            up = h[:, EIS:]
            inter = ((gate * jax.nn.sigmoid(gate)) * up).astype(jnp.bfloat16)
            o = jax.lax.dot_general(
                inter, dn_bf,
                dimension_numbers=(((1,), (0,)), ((), ())),
                preferred_element_type=jnp.float32,
            )
            og_ref[pl.ds(rbase, CH), :] = o.astype(jnp.bfloat16)

        @pl.when(psmem_ref[1, e] > 0)
        def _():
            chunk(0)

        @pl.when(psmem_ref[1, e] > CH)
        def _():
            chunk(1)

        @pl.when(e + NBUF < E)
        def _():
            start(e + NBUF)

        return carry

    jax.lax.fori_loop(0, E, expert_body, 0)

    out_ref[...] = jax.lax.dot_general(
        r_ref[...], og_ref[...],
        dimension_numbers=(((1,), (0,)), ((), ())),
        preferred_element_type=jnp.float32,
    )


@functools.partial(jax.jit, static_argnames=("interpret",))
def kernel(x, token_ids, mu, gate_up_proj, down_proj, mu_router_w, interpret=False):
    tid2d = token_ids.reshape(T, 1)
    return pl.pallas_call(
        _moe_kernel,
        in_specs=[
            pl.BlockSpec((T, 1), lambda: (0, 0)),
            pl.BlockSpec(memory_space=pltpu.MemorySpace.HBM),
            pl.BlockSpec(memory_space=pltpu.MemorySpace.HBM),
            pl.BlockSpec((E, H), lambda: (0, 0)),
            pl.BlockSpec(memory_space=pltpu.MemorySpace.HBM),
            pl.BlockSpec(memory_space=pltpu.MemorySpace.HBM),
        ],
        out_specs=pl.BlockSpec((T, H), lambda: (0, 0)),
        out_shape=jax.ShapeDtypeStruct((T, H), jnp.float32),
        scratch_shapes=[
            pltpu.VMEM((T, H), jnp.float32),
            pltpu.VMEM((T, H), jnp.float32),
            pltpu.VMEM((T, RP), jnp.bfloat16),
            pltpu.VMEM((RP, H), jnp.bfloat16),
            pltpu.VMEM((RP, H), jnp.bfloat16),
            pltpu.VMEM((NBUF, H, 2 * EIS), jnp.float32),
            pltpu.VMEM((NBUF, EIS, H), jnp.float32),
            pltpu.VMEM((2, E), jnp.int32),
            pltpu.SMEM((2, E), jnp.int32),
            pltpu.SemaphoreType.DMA((NBUF,)),
            pltpu.SemaphoreType.DMA((NBUF,)),
            pltpu.SemaphoreType.DMA,
            pltpu.SemaphoreType.DMA,
            pltpu.SemaphoreType.DMA,
        ],
        interpret=interpret,
    )(tid2d, x, mu, mu_router_w, gate_up_proj, down_proj)


# speedup vs baseline: 1.2665x; 1.2665x over previous
import functools

import jax
import jax.numpy as jnp
from jax.experimental import pallas as pl
from jax.experimental.pallas import tpu as pltpu

H = 768
I = 4096
E = 16
V = 32000
EIS = I // E
T = 512
CH = 256
RP = 1024
NBUF = 8


def _moe_kernel(tid_ref, x_ref, mu_ref, w_ref, gup_ref, dnp_ref, out_ref,
                r_ref, xg_ref, og_ref, gu_buf, dn_buf,
                plan_ref, psmem_ref, gu_sem, dn_sem, p_sem):
    def start(e):
        slot = jax.lax.rem(e, NBUF)
        pltpu.make_async_copy(
            gup_ref.at[e], gu_buf.at[slot], gu_sem.at[slot]).start()
        pltpu.make_async_copy(
            dnp_ref.at[e], dn_buf.at[slot], dn_sem.at[slot]).start()

    for e in range(NBUF):
        start(e)

    logits = jax.lax.dot_general(
        mu_ref[...], w_ref[...],
        dimension_numbers=(((1,), (1,)), ((), ())),
        preferred_element_type=jnp.float32,
    )
    tid = tid_ref[...]
    base_e = jnp.bitwise_and(jnp.clip(tid, 0, V - 1), E - 1)
    iota_e = jax.lax.broadcasted_iota(jnp.int32, (T, E), 1)
    onehot_f = (base_e == iota_e).astype(jnp.float32)
    combined = onehot_f * 10.0 + logits
    m = jnp.max(combined, axis=-1, keepdims=True)
    eid = jnp.min(jnp.where(combined == m, iota_e, E), axis=-1, keepdims=True)

    onehot = (iota_e == eid).astype(jnp.int32)
    iota_t = jax.lax.broadcasted_iota(jnp.int32, (T, E), 0)
    cum = onehot
    d = 1
    while d < T:
        rolled = pltpu.roll(cum, d, 0)
        cum = cum + jnp.where(iota_t >= d, rolled, 0)
        d *= 2
    rank = jnp.sum(cum * onehot, axis=-1, keepdims=True) - 1
    counts = cum[T - 1:T, :]
    ctp = jnp.bitwise_and(counts + 15, ~15)
    iota_l = jax.lax.broadcasted_iota(jnp.int32, (1, E), 1)
    coff = ctp
    d = 1
    while d < E:
        rolled = pltpu.roll(coff, d, 1)
        coff = coff + jnp.where(iota_l >= d, rolled, 0)
        d *= 2
    off = coff - ctp
    off_tok = jnp.sum(off * onehot, axis=-1, keepdims=True)
    row = off_tok + rank

    plan_ref[0:1, :] = jnp.right_shift(off, 4)
    plan_ref[1:2, :] = ctp
    p_cp = pltpu.make_async_copy(plan_ref, psmem_ref, p_sem)
    p_cp.start()

    iota_r = jax.lax.broadcasted_iota(jnp.int32, (T, RP), 1)
    r_ref[...] = (iota_r == row).astype(jnp.bfloat16)
    xg = jax.lax.dot_general(
        r_ref[...], x_ref[...].astype(jnp.bfloat16),
        dimension_numbers=(((0,), (0,)), ((), ())),
        preferred_element_type=jnp.float32,
    )
    xg_ref[...] = xg.astype(jnp.bfloat16)
    og_ref[...] = jnp.zeros((RP, H), jnp.bfloat16)
    p_cp.wait()

    def expert_body(e, carry):
        slot = jax.lax.rem(e, NBUF)
        pltpu.make_async_copy(
            gup_ref.at[e], gu_buf.at[slot], gu_sem.at[slot]).wait()
        pltpu.make_async_copy(
            dnp_ref.at[e], dn_buf.at[slot], dn_sem.at[slot]).wait()
        gu_bf = gu_buf[slot].astype(jnp.bfloat16)
        dn_bf = dn_buf[slot].astype(jnp.bfloat16)

        def chunk(s):
            rbase = (psmem_ref[0, e] + s * (CH // 16)) * 16
            xchunk = xg_ref[pl.ds(rbase, CH), :]
            h = jax.lax.dot_general(
                xchunk, gu_bf,
                dimension_numbers=(((1,), (0,)), ((), ())),
                preferred_element_type=jnp.float32,
            )
            gate = h[:, :EIS]
            up = h[:, EIS:]
            inter = ((gate * jax.nn.sigmoid(gate)) * up).astype(jnp.bfloat16)
            o = jax.lax.dot_general(
                inter, dn_bf,
                dimension_numbers=(((1,), (0,)), ((), ())),
                preferred_element_type=jnp.float32,
            )
            og_ref[pl.ds(rbase, CH), :] = o.astype(jnp.bfloat16)

        @pl.when(psmem_ref[1, e] > 0)
        def _():
            chunk(0)

        @pl.when(psmem_ref[1, e] > CH)
        def _():
            chunk(1)

        @pl.when(e + NBUF < E)
        def _():
            start(e + NBUF)

        return carry

    jax.lax.fori_loop(0, E, expert_body, 0)

    out_ref[...] = jax.lax.dot_general(
        r_ref[...], og_ref[...],
        dimension_numbers=(((1,), (0,)), ((), ())),
        preferred_element_type=jnp.float32,
    )


@functools.partial(jax.jit, static_argnames=("interpret",))
def kernel(x, token_ids, mu, gate_up_proj, down_proj, mu_router_w, interpret=False):
    tid2d = token_ids.reshape(T, 1)
    return pl.pallas_call(
        _moe_kernel,
        in_specs=[
            pl.BlockSpec((T, 1), lambda: (0, 0)),
            pl.BlockSpec((T, H), lambda: (0, 0)),
            pl.BlockSpec((T, H), lambda: (0, 0)),
            pl.BlockSpec((E, H), lambda: (0, 0)),
            pl.BlockSpec(memory_space=pltpu.MemorySpace.HBM),
            pl.BlockSpec(memory_space=pltpu.MemorySpace.HBM),
        ],
        out_specs=pl.BlockSpec((T, H), lambda: (0, 0)),
        out_shape=jax.ShapeDtypeStruct((T, H), jnp.float32),
        scratch_shapes=[
            pltpu.VMEM((T, RP), jnp.bfloat16),
            pltpu.VMEM((RP, H), jnp.bfloat16),
            pltpu.VMEM((RP, H), jnp.bfloat16),
            pltpu.VMEM((NBUF, H, 2 * EIS), jnp.float32),
            pltpu.VMEM((NBUF, EIS, H), jnp.float32),
            pltpu.VMEM((2, E), jnp.int32),
            pltpu.SMEM((2, E), jnp.int32),
            pltpu.SemaphoreType.DMA((NBUF,)),
            pltpu.SemaphoreType.DMA((NBUF,)),
            pltpu.SemaphoreType.DMA,
        ],
        interpret=interpret,
    )(tid2d, x, mu, mu_router_w, gate_up_proj, down_proj)
